# baseline (speedup 1.0000x reference)
import functools

import jax
import jax.numpy as jnp
from jax import lax
from jax.experimental import pallas as pl
from jax.experimental.pallas import tpu as pltpu
from jax.experimental.pallas import tpu_sc as plsc

_N_CH = 256
_N_TOK = 2048
_B = 8
_LANES = 16
_NC, _NS = 2, 16
_NW = _NC * _NS
_RPW = _N_TOK // _NW


def _bias_mm_body(l_ref, a_ref, out_ref):
    prod = lax.dot_general(
        l_ref[...], l_ref[...],
        dimension_numbers=(((1,), (1,)), ((), ())),
        preferred_element_type=jnp.float32,
    )
    out_ref[...] = a_ref[0, 0] * prod


def _scaled_bias(L_row, alpha):
    return pl.pallas_call(
        _bias_mm_body,
        out_shape=jax.ShapeDtypeStruct((_N_CH, _N_CH), jnp.float32),
    )(L_row, alpha.reshape(1, 1))


def _sc_double_gather(bias, idx):
    mesh = plsc.VectorSubcoreMesh(
        core_axis_name="c", subcore_axis_name="s",
        num_cores=_NC, num_subcores=_NS,
    )

    @functools.partial(
        pl.kernel,
        out_type=jax.ShapeDtypeStruct((_N_TOK, _N_TOK), jnp.float32),
        mesh=mesh,
        compiler_params=pltpu.CompilerParams(
            use_tc_tiling_on_sc=False, needs_layout_passes=False),
        scratch_types=[
            pltpu.VMEM((_N_TOK,), jnp.int32),
            pltpu.VMEM((_RPW,), jnp.int32),
            pltpu.VMEM((_RPW, _N_CH), jnp.float32),
            pltpu.VMEM((_N_TOK,), jnp.float32),
            pltpu.SemaphoreType.DMA,
        ],
    )
    def k(bias_hbm, idx_hbm, out_hbm, idxc_v, idxr_v, rows_v, obuf_v, sem):
        wid = lax.axis_index("s") * _NC + lax.axis_index("c")
        base = wid * _RPW
        pltpu.sync_copy(idx_hbm, idxc_v)
        pltpu.sync_copy(idx_hbm.at[pl.ds(base, _RPW)], idxr_v)
        pltpu.async_copy(bias_hbm.at[idxr_v], rows_v, sem).wait()

        def row_body(i, _):
            row_splat = jnp.full((_LANES,), i, jnp.int32)

            def chunk_body(c, _):
                cols = idxc_v[pl.ds(c * _LANES, _LANES)]
                obuf_v[pl.ds(c * _LANES, _LANES)] = plsc.load_gather(
                    rows_v, [row_splat, cols])
                return 0

            lax.fori_loop(0, _N_TOK // _LANES, chunk_body, 0)
            pltpu.sync_copy(obuf_v, out_hbm.at[base + i])
            return 0

        lax.fori_loop(0, _RPW, row_body, 0)

    return k(bias, idx)


def _add_body(attn_ref, bm_ref, mask_ref, out_ref):
    b = jnp.where(mask_ref[...] != 0, bm_ref[...], 0.0)
    out_ref[...] = attn_ref[...] + b[None]


def _fused_add(attn_logits, bmat, mask_i8):
    ti = 256
    return pl.pallas_call(
        _add_body,
        grid=(_N_TOK // ti, _B),
        in_specs=[
            pl.BlockSpec((1, ti, _N_TOK), lambda i, b: (b, i, 0)),
            pl.BlockSpec((ti, _N_TOK), lambda i, b: (i, 0)),
            pl.BlockSpec((ti, _N_TOK), lambda i, b: (i, 0)),
        ],
        out_specs=pl.BlockSpec((1, ti, _N_TOK), lambda i, b: (b, i, 0)),
        out_shape=jax.ShapeDtypeStruct((_B, _N_TOK, _N_TOK), jnp.float32),
    )(attn_logits, bmat, mask_i8)


def kernel(attn_logits, L_row, alpha, ch_tok_mask, ch_indices):
    bias = _scaled_bias(L_row, alpha)
    bmat = _sc_double_gather(bias, ch_indices)
    return _fused_add(attn_logits, bmat, ch_tok_mask.astype(jnp.int8))


# speedup vs baseline: 1.4304x; 1.4304x over previous
import functools

import jax
import jax.numpy as jnp
from jax import lax
from jax.experimental import pallas as pl
from jax.experimental.pallas import tpu as pltpu
from jax.experimental.pallas import tpu_sc as plsc

_N_CH = 256
_N_TOK = 2048
_B = 8
_NC, _NS = 2, 16
_NW = _NC * _NS
_RPW = _N_TOK // _NW
_TI = 256


def _bias_mm_body(l_ref, a_ref, out_ref):
    prod = lax.dot_general(
        l_ref[...], l_ref[...],
        dimension_numbers=(((1,), (1,)), ((), ())),
        preferred_element_type=jnp.float32,
    )
    out_ref[...] = a_ref[0, 0] * prod


def _scaled_bias(L_row, alpha):
    return pl.pallas_call(
        _bias_mm_body,
        out_shape=jax.ShapeDtypeStruct((_N_CH, _N_CH), jnp.float32),
    )(L_row, alpha.reshape(1, 1))


def _sc_row_gather(bias, idx):
    mesh = plsc.VectorSubcoreMesh(
        core_axis_name="c", subcore_axis_name="s",
        num_cores=_NC, num_subcores=_NS,
    )

    @functools.partial(
        pl.kernel,
        out_type=jax.ShapeDtypeStruct((_N_TOK, _N_CH), jnp.float32),
        mesh=mesh,
        compiler_params=pltpu.CompilerParams(
            use_tc_tiling_on_sc=False, needs_layout_passes=False),
        scratch_types=[
            pltpu.VMEM((_RPW,), jnp.int32),
            pltpu.VMEM((_RPW, _N_CH), jnp.float32),
            pltpu.SemaphoreType.DMA,
        ],
    )
    def k(bias_hbm, idx_hbm, out_hbm, idxr_v, rows_v, sem):
        wid = lax.axis_index("s") * _NC + lax.axis_index("c")
        base = wid * _RPW
        pltpu.sync_copy(idx_hbm.at[pl.ds(base, _RPW)], idxr_v)
        pltpu.async_copy(bias_hbm.at[idxr_v], rows_v, sem).wait()
        pltpu.sync_copy(rows_v, out_hbm.at[pl.ds(base, _RPW)])

    return k(bias, idx)


def _add_body(attn_ref, r_ref, idx_ref, mask_ref, out_ref, fb_ref):
    @pl.when(pl.program_id(1) == 0)
    def _():
        onehot = (idx_ref[...] == lax.broadcasted_iota(
            jnp.int32, (_N_TOK, _N_CH), 1)).astype(jnp.float32)
        fb_ref[...] = lax.dot_general(
            r_ref[...], onehot,
            dimension_numbers=(((1,), (1,)), ((), ())),
            preferred_element_type=jnp.float32,
        )

    b = jnp.where(mask_ref[...] != 0, fb_ref[...], 0.0)
    out_ref[...] = attn_ref[...] + b[None]


def _fused_add(attn_logits, rows, idx2d, mask_i8):
    return pl.pallas_call(
        _add_body,
        grid=(_N_TOK // _TI, _B),
        in_specs=[
            pl.BlockSpec((1, _TI, _N_TOK), lambda i, b: (b, i, 0)),
            pl.BlockSpec((_TI, _N_CH), lambda i, b: (i, 0)),
            pl.BlockSpec((_N_TOK, 1), lambda i, b: (0, 0)),
            pl.BlockSpec((_TI, _N_TOK), lambda i, b: (i, 0)),
        ],
        out_specs=pl.BlockSpec((1, _TI, _N_TOK), lambda i, b: (b, i, 0)),
        out_shape=jax.ShapeDtypeStruct((_B, _N_TOK, _N_TOK), jnp.float32),
        scratch_shapes=[pltpu.VMEM((_TI, _N_TOK), jnp.float32)],
    )(attn_logits, rows, idx2d, mask_i8)


def kernel(attn_logits, L_row, alpha, ch_tok_mask, ch_indices):
    bias = _scaled_bias(L_row, alpha)
    rows = _sc_row_gather(bias, ch_indices)
    return _fused_add(attn_logits, rows, ch_indices.reshape(_N_TOK, 1),
                      ch_tok_mask.astype(jnp.int8))


# speedup vs baseline: 1.5887x; 1.1107x over previous
import functools

import jax
import jax.numpy as jnp
from jax import lax
from jax.experimental import pallas as pl
from jax.experimental.pallas import tpu as pltpu
from jax.experimental.pallas import tpu_sc as plsc

_N_CH = 256
_N_TOK = 2048
_B = 8
_NC, _NS = 2, 16
_NW = _NC * _NS
_RPW = _N_TOK // _NW
_TI = 512


def _bias_mm_body(l_ref, a_ref, out_ref):
    prod = lax.dot_general(
        l_ref[...], l_ref[...],
        dimension_numbers=(((1,), (1,)), ((), ())),
        preferred_element_type=jnp.float32,
    )
    out_ref[...] = a_ref[0, 0] * prod


def _scaled_bias(L_row, alpha):
    return pl.pallas_call(
        _bias_mm_body,
        out_shape=jax.ShapeDtypeStruct((_N_CH, _N_CH), jnp.float32),
    )(L_row, alpha.reshape(1, 1))


def _sc_row_gather(bias, idx):
    mesh = plsc.VectorSubcoreMesh(
        core_axis_name="c", subcore_axis_name="s",
        num_cores=_NC, num_subcores=_NS,
    )

    @functools.partial(
        pl.kernel,
        out_type=jax.ShapeDtypeStruct((_N_TOK, _N_CH), jnp.float32),
        mesh=mesh,
        compiler_params=pltpu.CompilerParams(
            use_tc_tiling_on_sc=False, needs_layout_passes=False),
        scratch_types=[
            pltpu.VMEM((_RPW,), jnp.int32),
            pltpu.VMEM((_RPW, _N_CH), jnp.float32),
            pltpu.SemaphoreType.DMA,
        ],
    )
    def k(bias_hbm, idx_hbm, out_hbm, idxr_v, rows_v, sem):
        wid = lax.axis_index("s") * _NC + lax.axis_index("c")
        base = wid * _RPW
        pltpu.sync_copy(idx_hbm.at[pl.ds(base, _RPW)], idxr_v)
        pltpu.async_copy(bias_hbm.at[idxr_v], rows_v, sem).wait()
        pltpu.sync_copy(rows_v, out_hbm.at[pl.ds(base, _RPW)])

    return k(bias, idx)


def _add_body(attn_ref, r_ref, idx_ref, mask_ref, out_ref, fb_ref):
    @pl.when(pl.program_id(1) == 0)
    def _():
        onehot = (idx_ref[...] == lax.broadcasted_iota(
            jnp.int32, (_N_TOK, _N_CH), 1)).astype(jnp.float32)
        fb_ref[...] = lax.dot_general(
            r_ref[...], onehot,
            dimension_numbers=(((1,), (1,)), ((), ())),
            preferred_element_type=jnp.float32,
        )

    b = jnp.where(mask_ref[...] != 0, fb_ref[...], 0.0)
    out_ref[...] = attn_ref[...] + b[None]


def _fused_add(attn_logits, rows, idx2d, mask_i8):
    return pl.pallas_call(
        _add_body,
        grid=(_N_TOK // _TI, _B),
        in_specs=[
            pl.BlockSpec((1, _TI, _N_TOK), lambda i, b: (b, i, 0)),
            pl.BlockSpec((_TI, _N_CH), lambda i, b: (i, 0)),
            pl.BlockSpec((_N_TOK, 1), lambda i, b: (0, 0)),
            pl.BlockSpec((_TI, _N_TOK), lambda i, b: (i, 0)),
        ],
        out_specs=pl.BlockSpec((1, _TI, _N_TOK), lambda i, b: (b, i, 0)),
        out_shape=jax.ShapeDtypeStruct((_B, _N_TOK, _N_TOK), jnp.float32),
        scratch_shapes=[pltpu.VMEM((_TI, _N_TOK), jnp.float32)],
    )(attn_logits, rows, idx2d, mask_i8)


def kernel(attn_logits, L_row, alpha, ch_tok_mask, ch_indices):
    bias = _scaled_bias(L_row, alpha)
    rows = _sc_row_gather(bias, ch_indices)
    return _fused_add(attn_logits, rows, ch_indices.reshape(_N_TOK, 1),
                      ch_tok_mask.astype(jnp.int8))


# speedup vs baseline: 1.6256x; 1.0232x over previous
import functools

import jax
import jax.numpy as jnp
from jax import lax
from jax.experimental import pallas as pl
from jax.experimental.pallas import tpu as pltpu
from jax.experimental.pallas import tpu_sc as plsc

_N_CH = 256
_N_TOK = 2048
_B = 8
_NC, _NS = 2, 16
_NW = _NC * _NS
_RPW = _N_TOK // _NW
_TI = 1024


def _bias_mm_body(l_ref, a_ref, out_ref):
    prod = lax.dot_general(
        l_ref[...], l_ref[...],
        dimension_numbers=(((1,), (1,)), ((), ())),
        preferred_element_type=jnp.float32,
    )
    out_ref[...] = a_ref[0, 0] * prod


def _scaled_bias(L_row, alpha):
    return pl.pallas_call(
        _bias_mm_body,
        out_shape=jax.ShapeDtypeStruct((_N_CH, _N_CH), jnp.float32),
    )(L_row, alpha.reshape(1, 1))


def _sc_row_gather(bias, idx):
    mesh = plsc.VectorSubcoreMesh(
        core_axis_name="c", subcore_axis_name="s",
        num_cores=_NC, num_subcores=_NS,
    )

    @functools.partial(
        pl.kernel,
        out_type=jax.ShapeDtypeStruct((_N_TOK, _N_CH), jnp.float32),
        mesh=mesh,
        compiler_params=pltpu.CompilerParams(
            use_tc_tiling_on_sc=False, needs_layout_passes=False),
        scratch_types=[
            pltpu.VMEM((_RPW,), jnp.int32),
            pltpu.VMEM((_RPW, _N_CH), jnp.float32),
            pltpu.SemaphoreType.DMA,
        ],
    )
    def k(bias_hbm, idx_hbm, out_hbm, idxr_v, rows_v, sem):
        wid = lax.axis_index("s") * _NC + lax.axis_index("c")
        base = wid * _RPW
        pltpu.sync_copy(idx_hbm.at[pl.ds(base, _RPW)], idxr_v)
        pltpu.async_copy(bias_hbm.at[idxr_v], rows_v, sem).wait()
        pltpu.sync_copy(rows_v, out_hbm.at[pl.ds(base, _RPW)])

    return k(bias, idx)


def _add_body(attn_ref, r_ref, idx_ref, mask_ref, out_ref, fb_ref):
    @pl.when(pl.program_id(1) == 0)
    def _():
        onehot = (idx_ref[...] == lax.broadcasted_iota(
            jnp.int32, (_N_TOK, _N_CH), 1)).astype(jnp.float32)
        fb_ref[...] = lax.dot_general(
            r_ref[...], onehot,
            dimension_numbers=(((1,), (1,)), ((), ())),
            preferred_element_type=jnp.float32,
        )

    b = jnp.where(mask_ref[...] != 0, fb_ref[...], 0.0)
    out_ref[...] = attn_ref[...] + b[None]


def _fused_add(attn_logits, rows, idx2d, mask_i8):
    return pl.pallas_call(
        _add_body,
        grid=(_N_TOK // _TI, _B),
        in_specs=[
            pl.BlockSpec((1, _TI, _N_TOK), lambda i, b: (b, i, 0)),
            pl.BlockSpec((_TI, _N_CH), lambda i, b: (i, 0)),
            pl.BlockSpec((_N_TOK, 1), lambda i, b: (0, 0)),
            pl.BlockSpec((_TI, _N_TOK), lambda i, b: (i, 0)),
        ],
        out_specs=pl.BlockSpec((1, _TI, _N_TOK), lambda i, b: (b, i, 0)),
        out_shape=jax.ShapeDtypeStruct((_B, _N_TOK, _N_TOK), jnp.float32),
        scratch_shapes=[pltpu.VMEM((_TI, _N_TOK), jnp.float32)],
    )(attn_logits, rows, idx2d, mask_i8)


def kernel(attn_logits, L_row, alpha, ch_tok_mask, ch_indices):
    bias = _scaled_bias(L_row, alpha)
    rows = _sc_row_gather(bias, ch_indices)
    return _fused_add(attn_logits, rows, ch_indices.reshape(_N_TOK, 1),
                      ch_tok_mask.astype(jnp.int8))
